# baseline (speedup 1.0000x reference)
import functools

import jax
import jax.numpy as jnp
from jax import lax
from jax.experimental import pallas as pl
from jax.experimental.pallas import tpu as pltpu
from jax.experimental.pallas import tpu_sc as plsc

_HIDDEN = 768
_NP = 8
_B = 4
_S = 2048
_NS = 3
_NW = 32
_TPW = (_B * _S) // _NW
_C = 64
_NCH = _TPW // _C
_ROWS = _NP + _S
_OUT_ROWS = _B * _NS * _ROWS


def _prompt_tc(sp_ref, wm_ref, wv_ref, wl_ref, out_ref):
    dn = (((1,), (1,)), ((), ()))
    sp = sp_ref[...]
    mean = lax.dot_general(sp, wm_ref[...], dn,
                           preferred_element_type=jnp.float32,
                           precision=lax.Precision.HIGHEST)
    logv = lax.dot_general(sp, wv_ref[...], dn,
                           preferred_element_type=jnp.float32,
                           precision=lax.Precision.HIGHEST)
    std = jnp.exp(0.5 * logv)
    for i, sgn in enumerate((-1.0, 0.0, 1.0)):
        z = mean + sgn * std
        out_ref[i * _NP:(i + 1) * _NP, :] = lax.dot_general(
            z, wl_ref[...], dn,
            preferred_element_type=jnp.float32,
            precision=lax.Precision.HIGHEST)


def _patch_tc(src_ref, epp_ref, out_ref):
    del src_ref
    out_ref[...] = epp_ref[...]


def _sc_body(tok_hbm, wte_hbm, out_hbm,
             idx_v, rows0, rows1, gsem0, gsem1, wsem0, wsem1):
    c = lax.axis_index("c")
    s = lax.axis_index("s")
    wid = s * 2 + c
    b = wid // 8
    p = lax.rem(wid, 8)
    bufs = (rows0, rows1)
    gsems = (gsem0, gsem1)
    wsems = (wsem0, wsem1)
    pltpu.sync_copy(tok_hbm.at[pl.ds(wid * _NCH, _NCH)], idx_v)
    g = [None] * _NCH
    w = [[] for _ in range(_NCH)]
    g[0] = pltpu.async_copy(wte_hbm.at[idx_v.at[0]], bufs[0], gsems[0])
    for j in range(_NCH):
        bj = j % 2
        if j + 1 < _NCH:
            if j >= 1:
                for d in w[j - 1]:
                    d.wait()
            g[j + 1] = pltpu.async_copy(
                wte_hbm.at[idx_v.at[j + 1]], bufs[1 - bj], gsems[1 - bj])
        g[j].wait()
        src_base = p * _TPW + j * _C
        for s_i in range(_NS):
            out_base = (b * _NS + s_i) * _ROWS + _NP + src_base
            w[j].append(pltpu.async_copy(
                bufs[bj], out_hbm.at[pl.ds(out_base, _C)], wsems[bj]))
    for d in w[_NCH - 2]:
        d.wait()
    for d in w[_NCH - 1]:
        d.wait()


_sc_gather = functools.partial(
    pl.kernel,
    mesh=plsc.VectorSubcoreMesh(core_axis_name="c", subcore_axis_name="s"),
    out_type=jax.ShapeDtypeStruct((_OUT_ROWS, _HIDDEN), jnp.float32),
    scratch_types=[
        pltpu.VMEM((_NCH, _C), jnp.int32),
        pltpu.VMEM((_C, _HIDDEN), jnp.float32),
        pltpu.VMEM((_C, _HIDDEN), jnp.float32),
        pltpu.SemaphoreType.DMA,
        pltpu.SemaphoreType.DMA,
        pltpu.SemaphoreType.DMA,
        pltpu.SemaphoreType.DMA,
    ],
)(_sc_body)


def kernel(tokens, wte_weight, soft_prompt_embeds, W_mean, W_logv, W_l2h):
    epp = pl.pallas_call(
        _prompt_tc,
        out_shape=jax.ShapeDtypeStruct((_NS * _NP, _HIDDEN), jnp.float32),
    )(soft_prompt_embeds, W_mean, W_logv, W_l2h)
    tok = tokens.astype(jnp.int32).reshape(_NW * _NCH, _C)
    out = _sc_gather(tok, wte_weight)
    out = pl.pallas_call(
        _patch_tc,
        grid=(_B * _NS,),
        in_specs=[
            pl.BlockSpec((_NP, _HIDDEN), lambda i: (i * (_ROWS // _NP), 0)),
            pl.BlockSpec((_NP, _HIDDEN), lambda i: (i % _NS, 0)),
        ],
        out_specs=pl.BlockSpec((_NP, _HIDDEN), lambda i: (i * (_ROWS // _NP), 0)),
        out_shape=jax.ShapeDtypeStruct((_OUT_ROWS, _HIDDEN), jnp.float32),
        input_output_aliases={0: 0},
    )(out, epp)
    return out.reshape(_B * _NS, _ROWS, _HIDDEN)


# speedup vs baseline: 1.0964x; 1.0964x over previous
import functools

import jax
import jax.numpy as jnp
from jax import lax
from jax.experimental import pallas as pl
from jax.experimental.pallas import tpu as pltpu
from jax.experimental.pallas import tpu_sc as plsc

_HIDDEN = 768
_NP = 8
_B = 4
_S = 2048
_NS = 3
_NW = 32
_TPW = (_B * _S) // _NW
_C = 64
_NCH = _TPW // _C
_NBUF = 2
_ROWS = _NP + _S
_OUT_ROWS = _B * _NS * _ROWS


def _prompt_tc(sp_ref, wm_ref, wv_ref, wl_ref, out_ref):
    dn = (((1,), (1,)), ((), ()))
    sp = sp_ref[...]
    mean = lax.dot_general(sp, wm_ref[...], dn,
                           preferred_element_type=jnp.float32,
                           precision=lax.Precision.HIGHEST)
    logv = lax.dot_general(sp, wv_ref[...], dn,
                           preferred_element_type=jnp.float32,
                           precision=lax.Precision.HIGHEST)
    std = jnp.exp(0.5 * logv)
    for i, sgn in enumerate((-1.0, 0.0, 1.0)):
        z = mean + sgn * std
        out_ref[i * _NP:(i + 1) * _NP, :] = lax.dot_general(
            z, wl_ref[...], dn,
            preferred_element_type=jnp.float32,
            precision=lax.Precision.HIGHEST)


def _patch_tc(src_ref, epp_ref, out_ref, sem):
    del src_ref
    copies = []
    for i in range(_B * _NS):
        copies.append(pltpu.make_async_copy(
            epp_ref.at[pl.ds((i % _NS) * _NP, _NP)],
            out_ref.at[pl.ds(i * _ROWS, _NP)],
            sem))
    for d in copies:
        d.start()
    for d in copies:
        d.wait()


def _sc_body(tok_hbm, wte_hbm, out_hbm, idx_v, *bufs_and_sems):
    bufs = bufs_and_sems[:_NBUF]
    gsems = bufs_and_sems[_NBUF:2 * _NBUF]
    wsems = bufs_and_sems[2 * _NBUF:3 * _NBUF]
    c = lax.axis_index("c")
    s = lax.axis_index("s")
    wid = s * 2 + c
    b = wid // 8
    p = lax.rem(wid, 8)
    pltpu.sync_copy(tok_hbm.at[b, pl.ds(p * _TPW, _TPW)], idx_v)
    g = [None] * _NCH
    w = [[] for _ in range(_NCH)]
    for j in range(_NBUF - 1):
        g[j] = pltpu.async_copy(
            wte_hbm.at[idx_v.at[pl.ds(j * _C, _C)]], bufs[j], gsems[j])
    for j in range(_NCH):
        bj = j % _NBUF
        jn = j + _NBUF - 1
        if jn < _NCH:
            if j >= 1:
                for d in w[j - 1]:
                    d.wait()
            g[jn] = pltpu.async_copy(
                wte_hbm.at[idx_v.at[pl.ds(jn * _C, _C)]],
                bufs[jn % _NBUF], gsems[jn % _NBUF])
        g[j].wait()
        src_base = p * _TPW + j * _C
        for s_i in range(_NS):
            out_base = (b * _NS + s_i) * _ROWS + _NP + src_base
            w[j].append(pltpu.async_copy(
                bufs[bj], out_hbm.at[pl.ds(out_base, _C)], wsems[bj]))
    for j in range(max(0, _NCH - _NBUF), _NCH):
        for d in w[j]:
            d.wait()


_sc_gather = functools.partial(
    pl.kernel,
    mesh=plsc.VectorSubcoreMesh(core_axis_name="c", subcore_axis_name="s"),
    out_type=jax.ShapeDtypeStruct((_OUT_ROWS, _HIDDEN), jnp.float32),
    scratch_types=(
        [pltpu.VMEM((_TPW,), jnp.int32)]
        + [pltpu.VMEM((_C, _HIDDEN), jnp.float32)] * _NBUF
        + [pltpu.SemaphoreType.DMA] * (2 * _NBUF)
    ),
)(_sc_body)


def kernel(tokens, wte_weight, soft_prompt_embeds, W_mean, W_logv, W_l2h):
    epp = pl.pallas_call(
        _prompt_tc,
        out_shape=jax.ShapeDtypeStruct((_NS * _NP, _HIDDEN), jnp.float32),
    )(soft_prompt_embeds, W_mean, W_logv, W_l2h)
    out = _sc_gather(tokens.astype(jnp.int32), wte_weight)
    out = pl.pallas_call(
        _patch_tc,
        in_specs=[
            pl.BlockSpec(memory_space=pl.ANY),
            pl.BlockSpec((_NS * _NP, _HIDDEN), lambda: (0, 0)),
        ],
        out_specs=pl.BlockSpec(memory_space=pl.ANY),
        out_shape=jax.ShapeDtypeStruct((_OUT_ROWS, _HIDDEN), jnp.float32),
        scratch_shapes=[pltpu.SemaphoreType.DMA],
        input_output_aliases={0: 0},
    )(out, epp)
    return out.reshape(_B * _NS, _ROWS, _HIDDEN)
